# Initial kernel scaffold; baseline (speedup 1.0000x reference)
import jax
import jax.numpy as jnp
from jax.experimental import pallas as pl


def kernel(events, params):
    raise NotImplementedError("write your pallas kernel here")


# speedup vs baseline: 2.5300x; 2.5300x over previous
import functools

import jax
import jax.numpy as jnp
from jax.experimental import pallas as pl


def _mm_body(w_ref, x_ref, o_ref):
    o_ref[0] = jnp.dot(w_ref[...], x_ref[0],
                       preferred_element_type=jnp.float32)


def _mm_exact_body(w_ref, x_ref, o_ref):
    w = w_ref[...]
    x = x_ref[0]
    Ci = w.shape[1]
    acc = w[:, 0:1] * x[0:1, :]
    for i in range(1, Ci):
        acc = acc + w[:, i:i + 1] * x[i:i + 1, :]
    o_ref[0] = acc


def _bmm(x, w, exact=False):
    B, Ci, M = x.shape
    Co = w.shape[0]
    Tm = M if M <= 2048 else 2048
    grid = (B, M // Tm)
    return pl.pallas_call(
        _mm_exact_body if exact else _mm_body,
        grid=grid,
        in_specs=[
            pl.BlockSpec((Co, Ci), lambda b, i: (0, 0)),
            pl.BlockSpec((1, Ci, Tm), lambda b, i: (b, 0, i)),
        ],
        out_specs=pl.BlockSpec((1, Co, Tm), lambda b, i: (b, 0, i)),
        out_shape=jax.ShapeDtypeStruct((B, Co, M), jnp.float32),
    )(w, x)


def _fps_body(npoint, xt_ref, o_ref):
    x0 = xt_ref[0, 0:1, :]
    x1 = xt_ref[0, 1:2, :]
    x2 = xt_ref[0, 2:3, :]
    N = x0.shape[1]
    iota = jax.lax.broadcasted_iota(jnp.int32, (1, N), 1)
    iota_np = jax.lax.broadcasted_iota(jnp.int32, (1, npoint), 1)

    def step(i, state):
        dists, far, idxs = state
        idxs = jnp.where(iota_np == i, far, idxs)
        oh = iota == far
        cx = jnp.sum(jnp.where(oh, x0, 0.0))
        cy = jnp.sum(jnp.where(oh, x1, 0.0))
        cz = jnp.sum(jnp.where(oh, x2, 0.0))
        d = (x0 - cx) ** 2 + (x1 - cy) ** 2 + (x2 - cz) ** 2
        dists = jnp.minimum(dists, d)
        far = jnp.argmax(dists).astype(jnp.int32)
        return dists, far, idxs

    init = (jnp.full((1, N), 1e10, jnp.float32), jnp.int32(0),
            jnp.zeros((1, npoint), jnp.int32))
    _, _, idxs = jax.lax.fori_loop(0, npoint, step, init)
    o_ref[0] = idxs


def _fps(xyz_t, npoint):
    B, _, N = xyz_t.shape
    out = pl.pallas_call(
        functools.partial(_fps_body, npoint),
        grid=(B,),
        in_specs=[pl.BlockSpec((1, 3, N), lambda b: (b, 0, 0))],
        out_specs=pl.BlockSpec((1, 1, npoint), lambda b: (b, 0, 0)),
        out_shape=jax.ShapeDtypeStruct((B, 1, npoint), jnp.int32),
    )(xyz_t)
    return out[:, 0, :]


def _fps_xla(xyz, npoint):
    B, N, _ = xyz.shape
    def body(i, state):
        dists, farthest, idxs = state
        idxs = idxs.at[:, i].set(farthest)
        centroid = xyz[jnp.arange(B), farthest][:, None, :]
        d = jnp.sum((xyz - centroid) ** 2, axis=-1)
        dists = jnp.minimum(dists, d)
        farthest = jnp.argmax(dists, axis=-1).astype(jnp.int32)
        return (dists, farthest, idxs)
    state = (jnp.full((B, N), 1e10, jnp.float32), jnp.zeros((B,), jnp.int32),
             jnp.zeros((B, npoint), jnp.int32))
    _, _, idxs = jax.lax.fori_loop(0, npoint, body, state)
    return idxs


def _bq_body(r2, K, xyz_ref, nxt_ref, o_ref):
    N = xyz_ref.shape[1]
    T = nxt_ref.shape[2]
    d2 = None
    for c in range(3):
        a = xyz_ref[0, :, c:c + 1]
        bc = nxt_ref[0, c:c + 1, :]
        diff = a - bc
        sq = diff * diff
        d2 = sq if d2 is None else d2 + sq
    iota = jax.lax.broadcasted_iota(jnp.int32, (N, T), 0)
    cand = jnp.where(d2 <= r2, iota, jnp.int32(N))
    first = None
    for k in range(K):
        m = jnp.min(cand, axis=0, keepdims=True)
        if k == 0:
            first = m
        o_ref[0, k:k + 1, :] = jnp.where(m == N, first, m)
        cand = jnp.where(cand == m, jnp.int32(N), cand)


def _bq_xla(radius, nsample, xyz, new_xyz):
    B, N, _ = xyz.shape
    S = new_xyz.shape[1]
    sqrdists = jnp.sum((new_xyz[:, :, None, :] - xyz[:, None, :, :]) ** 2, axis=-1)
    group_idx = jnp.broadcast_to(jnp.arange(N, dtype=jnp.int32), (B, S, N))
    group_idx = jnp.where(sqrdists > radius ** 2, jnp.int32(N), group_idx)
    group_idx = jnp.sort(group_idx, axis=-1)[:, :, :nsample]
    first = group_idx[:, :, :1]
    group_idx = jnp.where(group_idx == N, first, group_idx)
    return group_idx


def _ball_query(radius, K, xyz, new_xyz_t, tile):
    B, N, _ = xyz.shape
    S = new_xyz_t.shape[2]
    T = min(tile, S)
    out = pl.pallas_call(
        functools.partial(_bq_body, radius * radius, K),
        grid=(B, S // T),
        in_specs=[
            pl.BlockSpec((1, N, 3), lambda b, i: (b, 0, 0)),
            pl.BlockSpec((1, 3, T), lambda b, i: (b, 0, i)),
        ],
        out_specs=pl.BlockSpec((1, K, T), lambda b, i: (b, 0, i)),
        out_shape=jax.ShapeDtypeStruct((B, K, S), jnp.int32),
    )(xyz, new_xyz_t)
    return jnp.transpose(out, (0, 2, 1))


def _fp_body(x2_ref, x1t_ref, p2_ref, o_ref):
    S2 = x2_ref.shape[1]
    T = x1t_ref.shape[2]
    d2 = None
    for c in range(3):
        a = x2_ref[0, :, c:c + 1]
        bc = x1t_ref[0, c:c + 1, :]
        diff = a - bc
        sq = diff * diff
        d2 = sq if d2 is None else d2 + sq
    d2 = jnp.maximum(d2, 1e-10)
    iota = jax.lax.broadcasted_iota(jnp.int32, (S2, T), 0)
    cur = d2
    vals, idxs = [], []
    for _ in range(3):
        m = jnp.min(cur, axis=0, keepdims=True)
        am = jnp.min(jnp.where(cur == m, iota, jnp.int32(S2)),
                     axis=0, keepdims=True)
        vals.append(m)
        idxs.append(am)
        cur = jnp.where(iota == am, jnp.float32(jnp.inf), cur)
    w = [1.0 / v for v in vals]
    wsum = w[0] + w[1] + w[2]
    Wd = None
    for k in range(3):
        wk = w[k] / wsum
        term = jnp.where(iota == idxs[k], wk, 0.0)
        Wd = term if Wd is None else Wd + term
    o_ref[0] = jnp.dot(p2_ref[0], Wd, preferred_element_type=jnp.float32,
                       precision=jax.lax.Precision.HIGHEST)


def _fp_interp(xyz2, xyz1_t, p2, tile):
    B, S2, _ = xyz2.shape
    S1 = xyz1_t.shape[2]
    C2 = p2.shape[1]
    T = min(tile, S1)
    return pl.pallas_call(
        _fp_body,
        grid=(B, S1 // T),
        in_specs=[
            pl.BlockSpec((1, S2, 3), lambda b, i: (b, 0, 0)),
            pl.BlockSpec((1, 3, T), lambda b, i: (b, 0, i)),
            pl.BlockSpec((1, C2, S2), lambda b, i: (b, 0, 0)),
        ],
        out_specs=pl.BlockSpec((1, C2, T), lambda b, i: (b, 0, i)),
        out_shape=jax.ShapeDtypeStruct((B, C2, S1), jnp.float32),
    )(xyz2, xyz1_t, p2)


def _index_points(points, idx):
    B = points.shape[0]
    batch = jnp.arange(B).reshape((B,) + (1,) * (idx.ndim - 1))
    return points[batch, idx]


def _bn_relu(h, eps=1e-5):
    m = jnp.mean(h, axis=(0, 2), keepdims=True)
    v = jnp.var(h, axis=(0, 2), keepdims=True)
    return jax.nn.relu((h - m) / jnp.sqrt(v + eps))


def _conv_bn_relu(x, conv, bn, stats4d=None):
    W, b = conv
    g, be = bn
    h = _bmm(x, W) + b[None, :, None]
    if stats4d is not None:
        S, K = stats4d
        h4 = h.reshape(h.shape[0], h.shape[1], S, K)
        m = jnp.mean(h4, axis=(0, 2, 3), keepdims=True)
        v = jnp.var(h4, axis=(0, 2, 3), keepdims=True)
        h4 = jax.nn.relu((h4 - m) / jnp.sqrt(v + 1e-5)
                         * g[None, :, None, None] + be[None, :, None, None])
        return h4.reshape(h.shape)
    m = jnp.mean(h, axis=(0, 2), keepdims=True)
    v = jnp.var(h, axis=(0, 2), keepdims=True)
    return jax.nn.relu((h - m) / jnp.sqrt(v + 1e-5) * g[None, :, None]
                       + be[None, :, None])


def _sa(xyz, feats, npoint, radius, nsample, p, bq_tile):
    B, N, _ = xyz.shape
    xyz_t = jnp.transpose(xyz, (0, 2, 1))
    fps_idx = _fps(xyz_t, npoint)
    new_xyz = _index_points(xyz, fps_idx)
    new_xyz_t = jnp.transpose(new_xyz, (0, 2, 1))
    idx = _ball_query(radius, nsample, xyz, new_xyz_t, bq_tile)
    grouped_xyz = _index_points(xyz, idx) - new_xyz[:, :, None, :]
    pts = jnp.transpose(feats, (0, 2, 1))
    grouped_pts = _index_points(pts, idx)
    grouped = jnp.concatenate([grouped_xyz, grouped_pts], axis=-1)
    x = jnp.transpose(grouped, (0, 3, 1, 2))
    S, K = npoint, nsample
    x = x.reshape(B, x.shape[1], S * K)
    for (W, b), (g, be) in zip(p['convs'], p['bns']):
        x = _conv_bn_relu(x, (W, b), (g, be))
    new_feats = jnp.max(x.reshape(B, x.shape[1], S, K), axis=-1)
    return new_xyz, new_feats


def _fp_interp_xla(xyz1, xyz2, points2):
    dists = jnp.sum((xyz1[:, :, None, :] - xyz2[:, None, :, :]) ** 2, axis=-1)
    dists = jnp.maximum(dists, 1e-10)
    k = min(3, xyz2.shape[1])
    neg_vals, idx = jax.lax.top_k(-dists, k)
    dist = -neg_vals
    w = 1.0 / dist
    w = w / jnp.sum(w, axis=2, keepdims=True)
    gathered = jax.vmap(lambda p2, ix: p2[:, ix])(points2, idx)
    return jnp.sum(gathered * w[:, None, :, :], axis=3)


def _fp(xyz1, xyz2, points1, points2, p, tile):
    xyz1_t = jnp.transpose(xyz1, (0, 2, 1))
    interpolated = _fp_interp(xyz2, xyz1_t, points2, tile)
    new = jnp.concatenate([points1, interpolated], axis=1)
    for (W, b), (g, be) in zip(p['convs'], p['bns']):
        new = _conv_bn_relu(new, (W, b), (g, be))
    return new


def _head(x, p, use_sigmoid):
    h = _conv_bn_relu(x, p['conv1'], p['bn1'])
    W2, b2 = p['conv2']
    h = _bmm(h, W2) + b2[None, :, None]
    if use_sigmoid:
        h = jax.nn.sigmoid(h)
    return h


@jax.jit
def kernel(events, params):
    xyz0 = events[:, :, :3]
    feats0 = jnp.transpose(events, (0, 2, 1))
    xyz1, f1 = _sa(xyz0, feats0, 512, 0.1, 32, params['sa1'], bq_tile=256)
    xyz2, f2 = _sa(xyz1, f1, 256, 0.2, 64, params['sa2'], bq_tile=256)
    f1_up = _fp(xyz1, xyz2, f1, f2, params['fp2'], tile=512)
    f0_up = _fp(xyz0, xyz1, feats0, f1_up, params['fp1'], tile=512)
    d = _head(f0_up, params['desc'], False)
    norm = jnp.sqrt(jnp.sum(d ** 2, axis=1, keepdims=True))
    d = d / jnp.maximum(norm, 1e-12)
    kp = _head(f0_up, params['kp'], True)
    unc = _head(f0_up, params['unc'], True)
    return (d, kp, unc)
